# baseline (speedup 1.0000x reference)
import functools

import jax
import jax.numpy as jnp
from jax import lax
from jax.experimental import pallas as pl
from jax.experimental.pallas import tpu as pltpu
from jax.experimental.pallas import tpu_sc as plsc

N = 10000
K = 32
C = 128
EPS = 1e-5
NKF = float(N * K)

NC = 2
NS = 16
L = 16
NW = NC * NS

IPW = (N * K) // NW
NV = IPW // L


@functools.cache
def _build_hist():
    mesh = plsc.VectorSubcoreMesh(
        core_axis_name="c", subcore_axis_name="s", num_cores=NC, num_subcores=NS
    )

    @functools.partial(
        pl.kernel,
        out_type=jax.ShapeDtypeStruct((NW, N), jnp.float32),
        mesh=mesh,
        scratch_types=[
            pltpu.VMEM((IPW,), jnp.int32),
            pltpu.VMEM((N,), jnp.float32),
        ],
        compiler_params=pltpu.CompilerParams(needs_layout_passes=False),
    )
    def _hist(idx_hbm, out_hbm, idx_v, hist_v):
        wid = lax.axis_index("s") * NC + lax.axis_index("c")
        zeros = jnp.zeros((L,), jnp.float32)

        def zero_body(i, carry):
            hist_v[pl.ds(i * L, L)] = zeros
            return carry

        lax.fori_loop(0, N // L, zero_body, 0)
        pltpu.sync_copy(idx_hbm.at[pl.ds(wid * IPW, IPW)], idx_v)
        ones = jnp.ones((L,), jnp.float32)

        def add_body(i, carry):
            ix = idx_v[pl.ds(i * L, L)]
            plsc.addupdate_scatter(hist_v, [ix], ones)
            return carry

        lax.fori_loop(0, NV, add_body, 0)
        pltpu.sync_copy(hist_v, out_hbm.at[wid])

    return _hist


NP_PAD = 10240
WTOT = C // 2
WPT = WTOT // NS
PPC = NP_PAD // NC
CHP = 256
NCHK = PPC // CHP
GPC = CHP // L


@functools.cache
def _build_gmax():
    mesh = plsc.VectorSubcoreMesh(
        core_axis_name="c", subcore_axis_name="s", num_cores=NC, num_subcores=NS
    )

    @functools.partial(
        pl.kernel,
        out_type=jax.ShapeDtypeStruct((WTOT, NP_PAD), jnp.int32),
        mesh=mesh,
        scratch_types=[
            pltpu.VMEM((WPT, N), jnp.int32),
            pltpu.VMEM((2, K, CHP), jnp.int32),
            pltpu.VMEM((WPT, PPC), jnp.int32),
            pltpu.SemaphoreType.DMA,
            pltpu.SemaphoreType.DMA,
        ],
        compiler_params=pltpu.CompilerParams(
            needs_layout_passes=False, use_tc_tiling_on_sc=False
        ),
    )
    def _gmax(reft_hbm, z2t_hbm, out_hbm, tbl_v, refc_v, out_v, sem0, sem1):
        cid = lax.axis_index("c")
        sid = lax.axis_index("s")
        base_pt = cid * PPC
        sems = (sem0, sem1)
        pltpu.sync_copy(z2t_hbm.at[pl.ds(sid * WPT, WPT)], tbl_v)

        def ref_start(ch, buf):
            pltpu.async_copy(
                reft_hbm.at[:, pl.ds(base_pt + ch * CHP, CHP)],
                refc_v.at[buf],
                sems[buf],
            )

        def ref_wait(buf):
            pltpu.make_async_copy(
                reft_hbm.at[:, pl.ds(0, CHP)], refc_v.at[buf], sems[buf]
            ).wait()

        ref_start(0, 0)
        wrows = [jnp.full((L,), w, jnp.int32) for w in range(WPT)]

        def outer(c2, carry):
            for b in (0, 1):
                ch = c2 * 2 + b

                @pl.when(ch + 1 < NCHK)
                def _():
                    ref_start(ch + 1, 1 - b)

                ref_wait(b)

                @plsc.parallel_loop(0, GPC)
                def grp(g):
                    col = g * L
                    acc = [[None] * 4 for _ in range(WPT)]
                    for k in range(K):
                        idxk = refc_v[b, k, pl.ds(col, L)]
                        for w in range(WPT):
                            x = plsc.bitcast(
                                plsc.load_gather(tbl_v, [wrows[w], idxk]),
                                jnp.bfloat16,
                            )
                            a = acc[w][k % 4]
                            acc[w][k % 4] = x if a is None else jnp.maximum(a, x)
                    for w in range(WPT):
                        m = jnp.maximum(
                            jnp.maximum(acc[w][0], acc[w][1]),
                            jnp.maximum(acc[w][2], acc[w][3]),
                        )
                        out_v[w, pl.ds(ch * CHP + col, L)] = plsc.bitcast(
                            m, jnp.int32
                        )
            return carry

        lax.fori_loop(0, NCHK // 2, outer, 0)
        pltpu.sync_copy(
            out_v, out_hbm.at[pl.ds(sid * WPT, WPT), pl.ds(base_pt, PPC)]
        )

    return _gmax


def _bn_cols(x, m, v, g, b):
    return (x - m) * lax.rsqrt(v + EPS) * g + b


def _tc_mid_body(feat_ref, wfc1_ref, gn1_ref, bn1_ref, wm1_ref, bm1_ref,
                 gm1_ref, bm1n_ref, wm2_ref, bm2_ref, gm2_ref, bm2n_ref,
                 cnt_ref, z2_ref):
    x = feat_ref[...]
    h = jnp.dot(x, wfc1_ref[...], preferred_element_type=jnp.float32)
    m = jnp.mean(h, axis=0, keepdims=True)
    v = jnp.mean((h - m) ** 2, axis=0, keepdims=True)
    h = jnp.maximum(_bn_cols(h, m, v, gn1_ref[...], bn1_ref[...]), 0.0)

    y1 = jnp.dot(h, wm1_ref[...], preferred_element_type=jnp.float32) + bm1_ref[...]
    cr = jnp.sum(cnt_ref[...], axis=0, keepdims=True)
    s1 = jnp.dot(cr, y1, preferred_element_type=jnp.float32) / NKF
    v1 = jnp.dot(cr, (y1 - s1) ** 2, preferred_element_type=jnp.float32) / NKF
    z1 = jnp.maximum(_bn_cols(y1, s1, v1, gm1_ref[...], bm1n_ref[...]), 0.0)

    y2 = jnp.dot(z1, wm2_ref[...], preferred_element_type=jnp.float32) + bm2_ref[...]
    s2 = jnp.dot(cr, y2, preferred_element_type=jnp.float32) / NKF
    v2 = jnp.dot(cr, (y2 - s2) ** 2, preferred_element_type=jnp.float32) / NKF
    z2 = jnp.maximum(_bn_cols(y2, s2, v2, gm2_ref[...], bm2n_ref[...]), 0.0)
    z2b = (z2 - jnp.max(z2, axis=0, keepdims=True)).astype(jnp.bfloat16)
    lo = lax.bitcast_convert_type(z2b[:, : C // 2], jnp.uint16).astype(jnp.uint32)
    hi = lax.bitcast_convert_type(z2b[:, C // 2 :], jnp.uint16).astype(jnp.uint32)
    words = lax.bitcast_convert_type(lo | (hi << 16), jnp.int32)
    z2_ref[...] = jnp.transpose(words, (1, 0))


def _tc_tail_body(pooled_ref, feat_ref, gn2_ref, bn2_ref, wfc3_ref,
                  gn3_ref, bn3_ref, out_ref):
    pw = lax.bitcast_convert_type(
        jnp.transpose(pooled_ref[...], (1, 0))[:N], jnp.uint32
    )
    lo = lax.bitcast_convert_type((pw & 0xFFFF).astype(jnp.uint16), jnp.bfloat16)
    hi = lax.bitcast_convert_type((pw >> 16).astype(jnp.uint16), jnp.bfloat16)
    p = jnp.concatenate(
        [lo.astype(jnp.float32), hi.astype(jnp.float32)], axis=1
    )
    m = jnp.mean(p, axis=0, keepdims=True)
    v = jnp.mean((p - m) ** 2, axis=0, keepdims=True)
    h = jnp.maximum(_bn_cols(p, m, v, gn2_ref[...], bn2_ref[...]), 0.0)
    o = jnp.dot(h, wfc3_ref[...], preferred_element_type=jnp.float32)
    m3 = jnp.mean(o, axis=0, keepdims=True)
    v3 = jnp.mean((o - m3) ** 2, axis=0, keepdims=True)
    out_ref[...] = jnp.maximum(
        feat_ref[...] + _bn_cols(o, m3, v3, gn3_ref[...], bn3_ref[...]), 0.0
    )


_tc_mid = pl.pallas_call(
    _tc_mid_body, out_shape=jax.ShapeDtypeStruct((C // 2, N), jnp.int32)
)
_tc_tail = pl.pallas_call(
    _tc_tail_body, out_shape=jax.ShapeDtypeStruct((N, C), jnp.float32)
)


def kernel(coord, feat, offset, reference_index, W_fc1, g_n1, b_n1, W_m1, b_m1,
           g_m1, b_m1n, W_m2, b_m2, g_m2, b_m2n, g_n2, b_n2, W_fc3, g_n3, b_n3):
    del coord, offset
    row = lambda a: a.reshape(1, C)
    idx_flat = reference_index.astype(jnp.int32).reshape(-1)

    cnt = _build_hist()(idx_flat)
    z2t_words = _tc_mid(feat, W_fc1, row(g_n1), row(b_n1), W_m1, row(b_m1),
                        row(g_m1), row(b_m1n), W_m2, row(b_m2), row(g_m2),
                        row(b_m2n), cnt)

    idx_pad = jnp.pad(idx_flat, (0, NP_PAD * K - N * K))
    reft = jnp.transpose(idx_pad.reshape(NP_PAD, K), (1, 0))
    pooledt_words = _build_gmax()(reft, z2t_words)
    return _tc_tail(pooledt_words, feat, row(g_n2), row(b_n2), W_fc3,
                    row(g_n3), row(b_n3))


# speedup vs baseline: 1.4177x; 1.4177x over previous
import functools

import jax
import jax.numpy as jnp
from jax import lax
from jax.experimental import pallas as pl
from jax.experimental.pallas import tpu as pltpu
from jax.experimental.pallas import tpu_sc as plsc

N = 10000
K = 32
C = 128
EPS = 1e-5
NKF = float(N * K)

NC = 2
NS = 16
L = 16
NW = NC * NS

IPW = (N * K) // NW
NV = IPW // L


@functools.cache
def _build_hist():
    mesh = plsc.VectorSubcoreMesh(
        core_axis_name="c", subcore_axis_name="s", num_cores=NC, num_subcores=NS
    )

    @functools.partial(
        pl.kernel,
        out_type=jax.ShapeDtypeStruct((NW, N), jnp.float32),
        mesh=mesh,
        scratch_types=[
            pltpu.VMEM((IPW,), jnp.int32),
            pltpu.VMEM((N,), jnp.float32),
        ],
        compiler_params=pltpu.CompilerParams(needs_layout_passes=False),
    )
    def _hist(idx_hbm, out_hbm, idx_v, hist_v):
        wid = lax.axis_index("s") * NC + lax.axis_index("c")
        zeros = jnp.zeros((L,), jnp.float32)

        def zero_body(i, carry):
            hist_v[pl.ds(i * L, L)] = zeros
            return carry

        lax.fori_loop(0, N // L, zero_body, 0)
        pltpu.sync_copy(idx_hbm.at[pl.ds(wid * IPW, IPW)], idx_v)
        ones = jnp.ones((L,), jnp.float32)

        def add_body(i, carry):
            ix = idx_v[pl.ds(i * L, L)]
            plsc.addupdate_scatter(hist_v, [ix], ones)
            return carry

        lax.fori_loop(0, NV, add_body, 0)
        pltpu.sync_copy(hist_v, out_hbm.at[wid])

    return _hist


NP_PAD = 10240
WTOT = C // 2
WPT = WTOT // NS
PPC = NP_PAD // NC
CHP = 256
NCHK = PPC // CHP
GPC = CHP // L


@functools.cache
def _build_gmax():
    mesh = plsc.VectorSubcoreMesh(
        core_axis_name="c", subcore_axis_name="s", num_cores=NC, num_subcores=NS
    )

    @functools.partial(
        pl.kernel,
        out_type=jax.ShapeDtypeStruct((WTOT, NP_PAD), jnp.int32),
        mesh=mesh,
        scratch_types=[
            pltpu.VMEM((WPT, N), jnp.int32),
            pltpu.VMEM((2, K, CHP), jnp.int32),
            pltpu.VMEM((WPT, PPC), jnp.int32),
            pltpu.SemaphoreType.DMA,
            pltpu.SemaphoreType.DMA,
        ],
        compiler_params=pltpu.CompilerParams(
            needs_layout_passes=False, use_tc_tiling_on_sc=False
        ),
    )
    def _gmax(reft_hbm, z2t_hbm, out_hbm, tbl_v, refc_v, out_v, sem0, sem1):
        cid = lax.axis_index("c")
        sid = lax.axis_index("s")
        base_pt = cid * PPC
        sems = (sem0, sem1)
        pltpu.sync_copy(z2t_hbm.at[pl.ds(sid * WPT, WPT)], tbl_v)

        def ref_start(ch, buf):
            pltpu.async_copy(
                reft_hbm.at[:, pl.ds(base_pt + ch * CHP, CHP)],
                refc_v.at[buf],
                sems[buf],
            )

        def ref_wait(buf):
            pltpu.make_async_copy(
                reft_hbm.at[:, pl.ds(0, CHP)], refc_v.at[buf], sems[buf]
            ).wait()

        ref_start(0, 0)
        wrows = [jnp.full((L,), w, jnp.int32) for w in range(WPT)]

        def outer(c2, carry):
            for b in (0, 1):
                ch = c2 * 2 + b

                @pl.when(ch + 1 < NCHK)
                def _():
                    ref_start(ch + 1, 1 - b)

                ref_wait(b)

                def grp(g, cc):
                    col = g * L
                    nch = 8
                    acc = [[None] * nch for _ in range(WPT)]
                    for k in range(K):
                        idxk = refc_v[b, k, pl.ds(col, L)]
                        for w in range(WPT):
                            x = plsc.bitcast(
                                plsc.load_gather(tbl_v, [wrows[w], idxk]),
                                jnp.bfloat16,
                            )
                            a = acc[w][k % nch]
                            acc[w][k % nch] = x if a is None else jnp.maximum(a, x)
                    for w in range(WPT):
                        t = acc[w]
                        while len(t) > 1:
                            t = [
                                jnp.maximum(t[2 * i], t[2 * i + 1])
                                for i in range(len(t) // 2)
                            ]
                        out_v[w, pl.ds(ch * CHP + col, L)] = plsc.bitcast(
                            t[0], jnp.int32
                        )
                    return cc

                lax.fori_loop(0, GPC, grp, 0)
            return carry

        lax.fori_loop(0, NCHK // 2, outer, 0)
        pltpu.sync_copy(
            out_v, out_hbm.at[pl.ds(sid * WPT, WPT), pl.ds(base_pt, PPC)]
        )

    return _gmax


def _bn_cols(x, m, v, g, b):
    return (x - m) * lax.rsqrt(v + EPS) * g + b


def _tc_mid_body(feat_ref, wfc1_ref, gn1_ref, bn1_ref, wm1_ref, bm1_ref,
                 gm1_ref, bm1n_ref, wm2_ref, bm2_ref, gm2_ref, bm2n_ref,
                 cnt_ref, z2_ref):
    x = feat_ref[...]
    h = jnp.dot(x, wfc1_ref[...], preferred_element_type=jnp.float32)
    m = jnp.mean(h, axis=0, keepdims=True)
    v = jnp.mean((h - m) ** 2, axis=0, keepdims=True)
    h = jnp.maximum(_bn_cols(h, m, v, gn1_ref[...], bn1_ref[...]), 0.0)

    y1 = jnp.dot(h, wm1_ref[...], preferred_element_type=jnp.float32) + bm1_ref[...]
    cr = jnp.sum(cnt_ref[...], axis=0, keepdims=True)
    s1 = jnp.dot(cr, y1, preferred_element_type=jnp.float32) / NKF
    v1 = jnp.dot(cr, (y1 - s1) ** 2, preferred_element_type=jnp.float32) / NKF
    z1 = jnp.maximum(_bn_cols(y1, s1, v1, gm1_ref[...], bm1n_ref[...]), 0.0)

    y2 = jnp.dot(z1, wm2_ref[...], preferred_element_type=jnp.float32) + bm2_ref[...]
    s2 = jnp.dot(cr, y2, preferred_element_type=jnp.float32) / NKF
    v2 = jnp.dot(cr, (y2 - s2) ** 2, preferred_element_type=jnp.float32) / NKF
    z2 = jnp.maximum(_bn_cols(y2, s2, v2, gm2_ref[...], bm2n_ref[...]), 0.0)
    z2b = (z2 - jnp.max(z2, axis=0, keepdims=True)).astype(jnp.bfloat16)
    lo = lax.bitcast_convert_type(z2b[:, : C // 2], jnp.uint16).astype(jnp.uint32)
    hi = lax.bitcast_convert_type(z2b[:, C // 2 :], jnp.uint16).astype(jnp.uint32)
    words = lax.bitcast_convert_type(lo | (hi << 16), jnp.int32)
    z2_ref[...] = jnp.transpose(words, (1, 0))


def _tc_tail_body(pooled_ref, feat_ref, gn2_ref, bn2_ref, wfc3_ref,
                  gn3_ref, bn3_ref, out_ref):
    pw = lax.bitcast_convert_type(
        jnp.transpose(pooled_ref[...], (1, 0))[:N], jnp.uint32
    )
    lo = lax.bitcast_convert_type((pw & 0xFFFF).astype(jnp.uint16), jnp.bfloat16)
    hi = lax.bitcast_convert_type((pw >> 16).astype(jnp.uint16), jnp.bfloat16)
    p = jnp.concatenate(
        [lo.astype(jnp.float32), hi.astype(jnp.float32)], axis=1
    )
    m = jnp.mean(p, axis=0, keepdims=True)
    v = jnp.mean((p - m) ** 2, axis=0, keepdims=True)
    h = jnp.maximum(_bn_cols(p, m, v, gn2_ref[...], bn2_ref[...]), 0.0)
    o = jnp.dot(h, wfc3_ref[...], preferred_element_type=jnp.float32)
    m3 = jnp.mean(o, axis=0, keepdims=True)
    v3 = jnp.mean((o - m3) ** 2, axis=0, keepdims=True)
    out_ref[...] = jnp.maximum(
        feat_ref[...] + _bn_cols(o, m3, v3, gn3_ref[...], bn3_ref[...]), 0.0
    )


_tc_mid = pl.pallas_call(
    _tc_mid_body, out_shape=jax.ShapeDtypeStruct((C // 2, N), jnp.int32)
)
_tc_tail = pl.pallas_call(
    _tc_tail_body, out_shape=jax.ShapeDtypeStruct((N, C), jnp.float32)
)


def kernel(coord, feat, offset, reference_index, W_fc1, g_n1, b_n1, W_m1, b_m1,
           g_m1, b_m1n, W_m2, b_m2, g_m2, b_m2n, g_n2, b_n2, W_fc3, g_n3, b_n3):
    del coord, offset
    row = lambda a: a.reshape(1, C)
    idx_flat = reference_index.astype(jnp.int32).reshape(-1)

    cnt = _build_hist()(idx_flat)
    z2t_words = _tc_mid(feat, W_fc1, row(g_n1), row(b_n1), W_m1, row(b_m1),
                        row(g_m1), row(b_m1n), W_m2, row(b_m2), row(g_m2),
                        row(b_m2n), cnt)

    idx_pad = jnp.pad(idx_flat, (0, NP_PAD * K - N * K))
    reft = jnp.transpose(idx_pad.reshape(NP_PAD, K), (1, 0))
    pooledt_words = _build_gmax()(reft, z2t_words)
    return _tc_tail(pooledt_words, feat, row(g_n2), row(b_n2), W_fc3,
                    row(g_n3), row(b_n3))


# speedup vs baseline: 1.4763x; 1.0414x over previous
import functools

import jax
import jax.numpy as jnp
from jax import lax
from jax.experimental import pallas as pl
from jax.experimental.pallas import tpu as pltpu
from jax.experimental.pallas import tpu_sc as plsc

N = 10000
K = 32
C = 128
EPS = 1e-5
NKF = float(N * K)

NC = 2
NS = 16
L = 16
NW = NC * NS

IPW = (N * K) // NW
NV = IPW // L


@functools.cache
def _build_hist():
    mesh = plsc.VectorSubcoreMesh(
        core_axis_name="c", subcore_axis_name="s", num_cores=NC, num_subcores=NS
    )

    @functools.partial(
        pl.kernel,
        out_type=jax.ShapeDtypeStruct((NW, N), jnp.float32),
        mesh=mesh,
        scratch_types=[
            pltpu.VMEM((IPW,), jnp.int32),
            pltpu.VMEM((N,), jnp.float32),
        ],
        compiler_params=pltpu.CompilerParams(needs_layout_passes=False),
    )
    def _hist(idx_hbm, out_hbm, idx_v, hist_v):
        wid = lax.axis_index("s") * NC + lax.axis_index("c")
        zeros = jnp.zeros((L,), jnp.float32)

        def zero_body(i, carry):
            hist_v[pl.ds(i * L, L)] = zeros
            return carry

        lax.fori_loop(0, N // L, zero_body, 0)
        pltpu.sync_copy(idx_hbm.at[pl.ds(wid * IPW, IPW)], idx_v)
        ones = jnp.ones((L,), jnp.float32)

        def add_body(i, carry):
            ix = idx_v[pl.ds(i * L, L)]
            plsc.addupdate_scatter(hist_v, [ix], ones)
            return carry

        lax.fori_loop(0, NV, add_body, 0)
        pltpu.sync_copy(hist_v, out_hbm.at[wid])

    return _hist


NP_PAD = 10240
WTOT = C // 2
WPT = WTOT // NS
PPC = NP_PAD // NC
CHP = 256
NCHK = PPC // CHP
GPC = CHP // L


@functools.cache
def _build_gmax():
    mesh = plsc.VectorSubcoreMesh(
        core_axis_name="c", subcore_axis_name="s", num_cores=NC, num_subcores=NS
    )

    @functools.partial(
        pl.kernel,
        out_type=jax.ShapeDtypeStruct((WTOT, NP_PAD), jnp.int32),
        mesh=mesh,
        scratch_types=[
            pltpu.VMEM((WPT, N), jnp.int32),
            pltpu.VMEM((2, K, CHP), jnp.int32),
            pltpu.VMEM((WPT, PPC), jnp.int32),
            pltpu.SemaphoreType.DMA,
            pltpu.SemaphoreType.DMA,
        ],
        compiler_params=pltpu.CompilerParams(
            needs_layout_passes=False, use_tc_tiling_on_sc=False
        ),
    )
    def _gmax(reft_hbm, z2t_hbm, out_hbm, tbl_v, refc_v, out_v, sem0, sem1):
        cid = lax.axis_index("c")
        sid = lax.axis_index("s")
        base_pt = cid * PPC
        sems = (sem0, sem1)
        pltpu.sync_copy(z2t_hbm.at[pl.ds(sid * WPT, WPT)], tbl_v)

        def ref_start(ch, buf):
            pltpu.async_copy(
                reft_hbm.at[:, pl.ds(base_pt + ch * CHP, CHP)],
                refc_v.at[buf],
                sems[buf],
            )

        def ref_wait(buf):
            pltpu.make_async_copy(
                reft_hbm.at[:, pl.ds(0, CHP)], refc_v.at[buf], sems[buf]
            ).wait()

        ref_start(0, 0)
        wrows = [jnp.full((L,), w, jnp.int32) for w in range(WPT)]

        def outer(c2, carry):
            for b in (0, 1):
                ch = c2 * 2 + b

                @pl.when(ch + 1 < NCHK)
                def _():
                    ref_start(ch + 1, 1 - b)

                ref_wait(b)

                def grp(g, cc):
                    col = g * L
                    nch = 4
                    acc = [[None] * nch for _ in range(WPT)]
                    for k in range(K):
                        idxk = refc_v[b, k, pl.ds(col, L)]
                        for w in range(WPT):
                            x = plsc.bitcast(
                                plsc.load_gather(tbl_v, [wrows[w], idxk]),
                                jnp.bfloat16,
                            )
                            a = acc[w][k % nch]
                            acc[w][k % nch] = x if a is None else jnp.maximum(a, x)
                    for w in range(WPT):
                        t = acc[w]
                        while len(t) > 1:
                            t = [
                                jnp.maximum(t[2 * i], t[2 * i + 1])
                                for i in range(len(t) // 2)
                            ]
                        out_v[w, pl.ds(ch * CHP + col, L)] = plsc.bitcast(
                            t[0], jnp.int32
                        )
                    return cc

                lax.fori_loop(0, GPC, grp, 0)
            return carry

        lax.fori_loop(0, NCHK // 2, outer, 0)
        pltpu.sync_copy(
            out_v, out_hbm.at[pl.ds(sid * WPT, WPT), pl.ds(base_pt, PPC)]
        )

    return _gmax


def _bn_cols(x, m, v, g, b):
    return (x - m) * lax.rsqrt(v + EPS) * g + b


def _tc_mid_body(feat_ref, wfc1_ref, gn1_ref, bn1_ref, wm1_ref, bm1_ref,
                 gm1_ref, bm1n_ref, wm2_ref, bm2_ref, gm2_ref, bm2n_ref,
                 cnt_ref, z2_ref):
    x = feat_ref[...]
    h = jnp.dot(x, wfc1_ref[...], preferred_element_type=jnp.float32)
    m = jnp.mean(h, axis=0, keepdims=True)
    v = jnp.mean((h - m) ** 2, axis=0, keepdims=True)
    h = jnp.maximum(_bn_cols(h, m, v, gn1_ref[...], bn1_ref[...]), 0.0)

    y1 = jnp.dot(h, wm1_ref[...], preferred_element_type=jnp.float32) + bm1_ref[...]
    cr = jnp.sum(cnt_ref[...], axis=0, keepdims=True)
    s1 = jnp.dot(cr, y1, preferred_element_type=jnp.float32) / NKF
    v1 = jnp.dot(cr, (y1 - s1) ** 2, preferred_element_type=jnp.float32) / NKF
    z1 = jnp.maximum(_bn_cols(y1, s1, v1, gm1_ref[...], bm1n_ref[...]), 0.0)

    y2 = jnp.dot(z1, wm2_ref[...], preferred_element_type=jnp.float32) + bm2_ref[...]
    s2 = jnp.dot(cr, y2, preferred_element_type=jnp.float32) / NKF
    v2 = jnp.dot(cr, (y2 - s2) ** 2, preferred_element_type=jnp.float32) / NKF
    z2 = jnp.maximum(_bn_cols(y2, s2, v2, gm2_ref[...], bm2n_ref[...]), 0.0)
    z2b = (z2 - jnp.max(z2, axis=0, keepdims=True)).astype(jnp.bfloat16)
    lo = lax.bitcast_convert_type(z2b[:, : C // 2], jnp.uint16).astype(jnp.uint32)
    hi = lax.bitcast_convert_type(z2b[:, C // 2 :], jnp.uint16).astype(jnp.uint32)
    words = lax.bitcast_convert_type(lo | (hi << 16), jnp.int32)
    z2_ref[...] = jnp.transpose(words, (1, 0))


def _tc_tail_body(pooled_ref, feat_ref, gn2_ref, bn2_ref, wfc3_ref,
                  gn3_ref, bn3_ref, out_ref):
    pw = lax.bitcast_convert_type(
        jnp.transpose(pooled_ref[...], (1, 0))[:N], jnp.uint32
    )
    lo = lax.bitcast_convert_type((pw & 0xFFFF).astype(jnp.uint16), jnp.bfloat16)
    hi = lax.bitcast_convert_type((pw >> 16).astype(jnp.uint16), jnp.bfloat16)
    p = jnp.concatenate(
        [lo.astype(jnp.float32), hi.astype(jnp.float32)], axis=1
    )
    m = jnp.mean(p, axis=0, keepdims=True)
    v = jnp.mean((p - m) ** 2, axis=0, keepdims=True)
    h = jnp.maximum(_bn_cols(p, m, v, gn2_ref[...], bn2_ref[...]), 0.0)
    o = jnp.dot(h, wfc3_ref[...], preferred_element_type=jnp.float32)
    m3 = jnp.mean(o, axis=0, keepdims=True)
    v3 = jnp.mean((o - m3) ** 2, axis=0, keepdims=True)
    out_ref[...] = jnp.maximum(
        feat_ref[...] + _bn_cols(o, m3, v3, gn3_ref[...], bn3_ref[...]), 0.0
    )


_tc_mid = pl.pallas_call(
    _tc_mid_body, out_shape=jax.ShapeDtypeStruct((C // 2, N), jnp.int32)
)
_tc_tail = pl.pallas_call(
    _tc_tail_body, out_shape=jax.ShapeDtypeStruct((N, C), jnp.float32)
)


def kernel(coord, feat, offset, reference_index, W_fc1, g_n1, b_n1, W_m1, b_m1,
           g_m1, b_m1n, W_m2, b_m2, g_m2, b_m2n, g_n2, b_n2, W_fc3, g_n3, b_n3):
    del coord, offset
    row = lambda a: a.reshape(1, C)
    idx_flat = reference_index.astype(jnp.int32).reshape(-1)

    cnt = _build_hist()(idx_flat)
    z2t_words = _tc_mid(feat, W_fc1, row(g_n1), row(b_n1), W_m1, row(b_m1),
                        row(g_m1), row(b_m1n), W_m2, row(b_m2), row(g_m2),
                        row(b_m2n), cnt)

    idx_pad = jnp.pad(idx_flat, (0, NP_PAD * K - N * K))
    reft = jnp.transpose(idx_pad.reshape(NP_PAD, K), (1, 0))
    pooledt_words = _build_gmax()(reft, z2t_words)
    return _tc_tail(pooledt_words, feat, row(g_n2), row(b_n2), W_fc3,
                    row(g_n3), row(b_n3))
